# Initial kernel scaffold; baseline (speedup 1.0000x reference)
import functools

import jax
import jax.numpy as jnp
from jax import lax
from jax.experimental import pallas as pl
from jax.experimental.pallas import tpu as pltpu
from jax.experimental.pallas import tpu_sc as plsc

_N = 50000
_E = 800000
_EB = 1000
_NB = 2000


def _relu(v):
    return jnp.maximum(v, 0.0)


def _dot(a, b):
    return lax.dot_general(a, b, (((1,), (0,)), ((), ())),
                           preferred_element_type=jnp.float32)


def _dot_t(w, h):
    return lax.dot_general(w, h, (((0,), (1,)), ((), ())),
                           preferred_element_type=jnp.float32)


def _dot_ct(a_t, w):
    return lax.dot_general(a_t, w, (((0,), (0,)), ((), ())),
                           preferred_element_type=jnp.float32)


def _k1_body(xi_ref, xj_ref, ea_ref, w0_ref, b0_ref, w1_ref, b1_ref,
             w2_ref, b2t_ref, out_ref):
    cat = jnp.concatenate([xi_ref[...], xj_ref[...], ea_ref[...]], axis=1)
    h = _relu(_dot(cat, w0_ref[...]) + b0_ref[...])
    h = _relu(_dot(h, w1_ref[...]) + b1_ref[...])
    out_ref[...] = _dot_t(w2_ref[...], h) + b2t_ref[...]


def _edge_mlp1(xg, ea, w0, b0, w1, b1, w2, b2):
    nblk = _E // _EB
    return pl.pallas_call(
        _k1_body,
        grid=(nblk,),
        in_specs=[
            pl.BlockSpec((_EB, 4), lambda i: (i, 0)),
            pl.BlockSpec((_EB, 4), lambda i, n=nblk: (i + n, 0)),
            pl.BlockSpec((_EB, 4), lambda i: (i, 0)),
            pl.BlockSpec((12, 64), lambda i: (0, 0)),
            pl.BlockSpec((1, 64), lambda i: (0, 0)),
            pl.BlockSpec((64, 64), lambda i: (0, 0)),
            pl.BlockSpec((1, 64), lambda i: (0, 0)),
            pl.BlockSpec((64, 64), lambda i: (0, 0)),
            pl.BlockSpec((64, 1), lambda i: (0, 0)),
        ],
        out_specs=pl.BlockSpec((64, _EB), lambda i: (0, i)),
        out_shape=jax.ShapeDtypeStruct((64, _E), jnp.float32),
    )(xg, xg, ea, w0, b0.reshape(1, 64), w1, b1.reshape(1, 64), w2,
      b2.reshape(64, 1))


def _k2_body(at_ref, x_ref, ga_ref, gx_ref, g0_ref, g1w_ref, g1b_ref,
             g2w_ref, g2b_ref, wd2_ref, b02_ref, ws2_ref, h_ref, t_ref):
    a = at_ref[...]
    a = jnp.where(a == -jnp.inf, 0.0, a)
    h = _relu(_dot_ct(a, ga_ref[...]) + _dot(x_ref[...], gx_ref[...])
              + g0_ref[...])
    h = _relu(_dot(h, g1w_ref[...]) + g1b_ref[...])
    h = _relu(_dot(h, g2w_ref[...]) + g2b_ref[...])
    h_ref[...] = h
    t_ref[0] = _dot(h, wd2_ref[...]) + b02_ref[...]
    t_ref[1] = _dot(h, ws2_ref[...])


def _gamma1_proj(a1t, x, ga, gx, g0, g1w, g1b, g2w, g2b, wd2, b02, ws2):
    nblk = _N // _NB
    return pl.pallas_call(
        _k2_body,
        grid=(nblk,),
        in_specs=[
            pl.BlockSpec((64, _NB), lambda i: (0, i)),
            pl.BlockSpec((_NB, 4), lambda i: (i, 0)),
            pl.BlockSpec((64, 64), lambda i: (0, 0)),
            pl.BlockSpec((4, 64), lambda i: (0, 0)),
            pl.BlockSpec((1, 64), lambda i: (0, 0)),
            pl.BlockSpec((64, 64), lambda i: (0, 0)),
            pl.BlockSpec((1, 64), lambda i: (0, 0)),
            pl.BlockSpec((64, 64), lambda i: (0, 0)),
            pl.BlockSpec((1, 64), lambda i: (0, 0)),
            pl.BlockSpec((64, 64), lambda i: (0, 0)),
            pl.BlockSpec((1, 64), lambda i: (0, 0)),
            pl.BlockSpec((64, 64), lambda i: (0, 0)),
        ],
        out_specs=[
            pl.BlockSpec((_NB, 64), lambda i: (i, 0)),
            pl.BlockSpec((2, _NB, 64), lambda i: (0, i, 0)),
        ],
        out_shape=[
            jax.ShapeDtypeStruct((_N, 64), jnp.float32),
            jax.ShapeDtypeStruct((2, _N, 64), jnp.float32),
        ],
    )(a1t, x, ga, gx, g0.reshape(1, 64), g1w, g1b.reshape(1, 64), g2w,
      g2b.reshape(1, 64), wd2, b02.reshape(1, 64), ws2)


def _k3_body(gd_ref, gs_ref, ea_ref, we_ref, w1_ref, b1_ref, w2_ref,
             b2t_ref, out_ref):
    h = _relu(gd_ref[...] + gs_ref[...] + _dot(ea_ref[...], we_ref[...]))
    h = _relu(_dot(h, w1_ref[...]) + b1_ref[...])
    out_ref[...] = _dot_t(w2_ref[...], h) + b2t_ref[...]


def _edge_mlp2(g2, ea, we, w1, b1, w2, b2):
    nblk = _E // _EB
    return pl.pallas_call(
        _k3_body,
        grid=(nblk,),
        in_specs=[
            pl.BlockSpec((_EB, 64), lambda i: (i, 0)),
            pl.BlockSpec((_EB, 64), lambda i, n=nblk: (i + n, 0)),
            pl.BlockSpec((_EB, 4), lambda i: (i, 0)),
            pl.BlockSpec((4, 64), lambda i: (0, 0)),
            pl.BlockSpec((64, 64), lambda i: (0, 0)),
            pl.BlockSpec((1, 64), lambda i: (0, 0)),
            pl.BlockSpec((64, 64), lambda i: (0, 0)),
            pl.BlockSpec((64, 1), lambda i: (0, 0)),
        ],
        out_specs=pl.BlockSpec((64, _EB), lambda i: (0, i)),
        out_shape=jax.ShapeDtypeStruct((64, _E), jnp.float32),
    )(g2, g2, ea, we, w1, b1.reshape(1, 64), w2, b2.reshape(64, 1))


def _k4_body(at_ref, h_ref, x_ref, goal_ref, ga_ref, gh_ref, g0_ref,
             g1w_ref, g1b_ref, g2w_ref, g2b_ref, qf_ref, qg_ref, q0_ref,
             q1w_ref, q1b_ref, q2w_ref, q2b_ref, out_ref):
    a = at_ref[...]
    a = jnp.where(a == -jnp.inf, 0.0, a)
    f = _relu(_dot_ct(a, ga_ref[...]) + _dot(h_ref[...], gh_ref[...])
              + g0_ref[...])
    f = _relu(_dot(f, g1w_ref[...]) + g1b_ref[...])
    feat = _dot(f, g2w_ref[...]) + g2b_ref[...]
    g = _relu(_dot(feat, qf_ref[...]) + _dot(goal_ref[...], qg_ref[...])
              + q0_ref[...])
    g = _relu(_dot(g, q1w_ref[...]) + q1b_ref[...])
    g = _dot(g, q2w_ref[...]) + q2b_ref[...]
    gains = 2.0 / (1.0 + jnp.exp(-g)) + 0.2
    x = x_ref[...]
    goal = goal_ref[...]
    s0 = x[:, 0:1] - goal[:, 0:1]
    s1 = x[:, 1:2] - goal[:, 1:2]
    ax = -(gains[:, 0:1] * s0 + gains[:, 1:2] * x[:, 2:3])
    ay = -(gains[:, 2:3] * s1 + gains[:, 3:4] * x[:, 3:4])
    out_ref[...] = jnp.concatenate([ax, ay], axis=1)


def _head(a2t, h, x, goal, ga, gh, g0, g1w, g1b, g2w, g2b,
          qf, qg, q0, q1w, q1b, q2w, q2b):
    nblk = _N // _NB

    def full(shape):
        return pl.BlockSpec(shape, lambda i, r=len(shape): (0,) * r)

    return pl.pallas_call(
        _k4_body,
        grid=(nblk,),
        in_specs=[
            pl.BlockSpec((64, _NB), lambda i: (0, i)),
            pl.BlockSpec((_NB, 64), lambda i: (i, 0)),
            pl.BlockSpec((_NB, 4), lambda i: (i, 0)),
            pl.BlockSpec((_NB, 8), lambda i: (i, 0)),
            full((64, 64)), full((64, 64)), full((1, 64)),
            full((64, 64)), full((1, 64)), full((64, 64)), full((1, 64)),
            full((64, 64)), full((8, 64)), full((1, 64)),
            full((64, 64)), full((1, 64)), full((64, 4)), full((1, 4)),
        ],
        out_specs=pl.BlockSpec((_NB, 2), lambda i: (i, 0)),
        out_shape=jax.ShapeDtypeStruct((_N, 2), jnp.float32),
    )(a2t, h, x, goal, ga, gh, g0.reshape(1, 64), g1w, g1b.reshape(1, 64),
      g2w, g2b.reshape(1, 64), qf, qg, q0.reshape(1, 64), q1w,
      q1b.reshape(1, 64), q2w, q2b.reshape(1, 4))


def _sc_gather(table, idx, win):
    m = idx.shape[0]
    d = table.shape[1]
    info = plsc.get_sparse_core_info()
    nc, ns = info.num_cores, info.num_subcores
    per_w = m // (nc * ns)
    nwin = per_w // win
    mesh = plsc.VectorSubcoreMesh(core_axis_name="c", subcore_axis_name="s")

    @functools.partial(
        pl.kernel,
        out_type=jax.ShapeDtypeStruct((m, d), jnp.float32),
        mesh=mesh,
        scratch_types=[
            pltpu.VMEM((win,), jnp.int32),
            pltpu.VMEM((win, d), jnp.float32),
            pltpu.SemaphoreType.DMA,
        ],
    )
    def gather_k(table_hbm, idx_hbm, out_hbm, idx_v, rows_v, sem):
        wid = lax.axis_index("s") * nc + lax.axis_index("c")
        base = wid * per_w

        def body(t, carry):
            off = base + t * win
            pltpu.sync_copy(idx_hbm.at[pl.ds(off, win)], idx_v)
            pltpu.async_copy(table_hbm.at[idx_v], rows_v, sem).wait()
            pltpu.sync_copy(rows_v, out_hbm.at[pl.ds(off, win)])
            return carry

        lax.fori_loop(0, nwin, body, 0)

    return gather_k(table, idx)


def _sc_segmax(msg_t, dst, win):
    e = msg_t.shape[1]
    nwin = e // win
    info = plsc.get_sparse_core_info()
    nc, ns = info.num_cores, info.num_subcores
    mesh = plsc.VectorSubcoreMesh(core_axis_name="c", subcore_axis_name="s")

    @functools.partial(
        pl.kernel,
        out_type=jax.ShapeDtypeStruct((64, _N), jnp.float32),
        mesh=mesh,
        scratch_types=[
            pltpu.VMEM((_N,), jnp.float32),
            pltpu.VMEM((_N,), jnp.float32),
            pltpu.VMEM((win,), jnp.int32),
            pltpu.VMEM((win,), jnp.float32),
            pltpu.VMEM((win,), jnp.float32),
        ],
    )
    def seg_k(msg_hbm, dst_hbm, out_hbm, acc0, acc1, idx_v, v0, v1):
        wid = lax.axis_index("s") * nc + lax.axis_index("c")
        r0 = 2 * wid
        neg = jnp.full((16,), -jnp.inf, jnp.float32)

        def init(i, carry):
            acc0[pl.ds(i * 16, 16)] = neg
            acc1[pl.ds(i * 16, 16)] = neg
            return carry

        lax.fori_loop(0, _N // 16, init, 0)

        def window(t, carry):
            off = t * win
            pltpu.sync_copy(dst_hbm.at[pl.ds(off, win)], idx_v)
            pltpu.sync_copy(msg_hbm.at[r0, pl.ds(off, win)], v0)
            pltpu.sync_copy(msg_hbm.at[r0 + 1, pl.ds(off, win)], v1)

            def group(j, gcarry):
                idx = idx_v[pl.ds(j * 16, 16)]
                a = v0[pl.ds(j * 16, 16)]
                b = v1[pl.ds(j * 16, 16)]

                def rmw(go):
                    c0 = plsc.load_gather(acc0, [idx])
                    c1 = plsc.load_gather(acc1, [idx])
                    p0 = a > c0
                    p1 = b > c1
                    plsc.store_scatter(acc0, [idx], a, mask=p0)
                    plsc.store_scatter(acc1, [idx], b, mask=p1)
                    return jnp.any(jnp.logical_or(p0, p1))

                lax.while_loop(lambda go: go, rmw, jnp.bool_(True))
                return gcarry

            lax.fori_loop(0, win // 16, group, 0)
            return carry

        lax.fori_loop(0, nwin, window, 0)
        pltpu.sync_copy(acc0, out_hbm.at[r0])
        pltpu.sync_copy(acc1, out_hbm.at[r0 + 1])

    return seg_k(msg_t, dst)


def kernel(x, edge_attr, edge_index, goal, params):
    src = edge_index[0]
    dst = edge_index[1]

    (w0, b0), (w1, b1), (w2, b2) = params['phi1']
    (g10, g1b0), (g11, g1b1), (g12, g1b2) = params['gamma1']
    (p0, pb0), (p1, pb1), (p2, pb2) = params['phi2']
    (g20, g2b0), (g21, g2b1), (g22, g2b2) = params['gamma2']
    (q0, qb0), (q1, qb1), (q2, qb2) = params['gains']

    idx1 = jnp.concatenate([dst, src])
    xg = _sc_gather(x, idx1, win=5000)
    msg1_t = _edge_mlp1(xg, edge_attr, w0, b0, w1, b1, w2, b2)
    a1_t = _sc_segmax(msg1_t, dst, win=4000)
    h, t = _gamma1_proj(a1_t, x, g10[:64], g10[64:68], g1b0, g11, g1b1,
                        g12, g1b2, p0[:64], pb0, p0[64:128])

    idx2 = jnp.concatenate([dst, src + _N])
    g2 = _sc_gather(t.reshape(2 * _N, 64), idx2, win=1000)
    msg2_t = _edge_mlp2(g2, edge_attr, p0[128:132], p1, pb1, p2, pb2)
    a2_t = _sc_segmax(msg2_t, dst, win=4000)
    return _head(a2_t, h, x, goal, g20[:64], g20[64:128], g2b0, g21, g2b1,
                 g22, g2b2, q0[:64], q0[64:72], qb0, q1, qb1, q2, qb2)


# speedup vs baseline: 1.4481x; 1.4481x over previous
import functools

import jax
import jax.numpy as jnp
from jax import lax
from jax.experimental import pallas as pl
from jax.experimental.pallas import tpu as pltpu
from jax.experimental.pallas import tpu_sc as plsc

_N = 50000
_E = 800000
_EB = 4096
_NB = 2048
_NS = 391


def _relu(v):
    return jnp.maximum(v, 0.0)


def _dot(a, b):
    return lax.dot_general(a, b, (((1,), (0,)), ((), ())),
                           preferred_element_type=jnp.float32)


def _dot_t(w, h):
    return lax.dot_general(w, h, (((0,), (1,)), ((), ())),
                           preferred_element_type=jnp.float32)


def _dot_ct(a_t, w):
    return lax.dot_general(a_t, w, (((0,), (0,)), ((), ())),
                           preferred_element_type=jnp.float32)


def _full_spec(shape):
    return pl.BlockSpec(shape, lambda i, r=len(shape): (0,) * r)


def _permute(v, idx):
    return lax.gather(
        v, idx[:, None],
        lax.GatherDimensionNumbers(offset_dims=(), collapsed_slice_dims=(0,),
                                   start_index_map=(0,)),
        (1,), mode=lax.GatherScatterMode.PROMISE_IN_BOUNDS)


def _prep_body(x_ref, wd_ref, ws_ref, b0_ref, t_ref):
    x = x_ref[...]
    t_ref[...] = jnp.concatenate(
        [_dot(x, wd_ref[...]) + b0_ref[...], _dot(x, ws_ref[...])], axis=1)


def _prep_table1(x, wd, ws, b0):
    nblk = (_N + _NB - 1) // _NB
    return pl.pallas_call(
        _prep_body,
        grid=(nblk,),
        in_specs=[
            pl.BlockSpec((_NB, 4), lambda i: (i, 0)),
            _full_spec((4, 64)), _full_spec((4, 64)), _full_spec((1, 64)),
        ],
        out_specs=pl.BlockSpec((_NB, 128), lambda i: (i, 0)),
        out_shape=jax.ShapeDtypeStruct((_N, 128), jnp.float32),
    )(x, wd, ws, b0.reshape(1, 64))


def _edge_body(g_ref, ea_ref, we_ref, w1_ref, b1_ref, w2_ref, b2t_ref,
               out_ref):
    h = _relu(g_ref[...] + _dot(ea_ref[...], we_ref[...]))
    h = _relu(_dot(h, w1_ref[...]) + b1_ref[...])
    m = _dot_t(w2_ref[...], h) + b2t_ref[...]
    out_ref[...] = m.reshape(64, _EB // 128, 128)


def _edge_mlp(g, ea, we, w1, b1, w2, b2):
    nblk = (_E + _EB - 1) // _EB
    return pl.pallas_call(
        _edge_body,
        grid=(nblk,),
        in_specs=[
            pl.BlockSpec((_EB, 64), lambda i: (i, 0)),
            pl.BlockSpec((_EB, 4), lambda i: (i, 0)),
            _full_spec((4, 64)),
            _full_spec((64, 64)), _full_spec((1, 64)),
            _full_spec((64, 64)), _full_spec((64, 1)),
        ],
        out_specs=pl.BlockSpec((64, _EB // 128, 128), lambda i: (0, i, 0)),
        out_shape=jax.ShapeDtypeStruct((64, _E // 128, 128), jnp.float32),
    )(g, ea, we, w1, b1.reshape(1, 64), w2, b2.reshape(64, 1))


def _k2_body(at_ref, x_ref, ga_ref, gx_ref, g0_ref, g1w_ref, g1b_ref,
             g2w_ref, g2b_ref, wd2_ref, b02_ref, ws2_ref, h_ref, t_ref):
    a = at_ref[...].reshape(64, _NB)
    a = jnp.where(a == -jnp.inf, 0.0, a)
    h = _relu(_dot_ct(a, ga_ref[...]) + _dot(x_ref[...], gx_ref[...])
              + g0_ref[...])
    h = _relu(_dot(h, g1w_ref[...]) + g1b_ref[...])
    h = _relu(_dot(h, g2w_ref[...]) + g2b_ref[...])
    h_ref[...] = h
    t_ref[...] = jnp.concatenate(
        [_dot(h, wd2_ref[...]) + b02_ref[...], _dot(h, ws2_ref[...])],
        axis=1)


def _gamma1_proj(a1t, x, ga, gx, g0, g1w, g1b, g2w, g2b, wd2, b02, ws2):
    nblk = (_N + _NB - 1) // _NB
    return pl.pallas_call(
        _k2_body,
        grid=(nblk,),
        in_specs=[
            pl.BlockSpec((64, _NB // 128, 128), lambda i: (0, i, 0)),
            pl.BlockSpec((_NB, 4), lambda i: (i, 0)),
            _full_spec((64, 64)), _full_spec((4, 64)), _full_spec((1, 64)),
            _full_spec((64, 64)), _full_spec((1, 64)),
            _full_spec((64, 64)), _full_spec((1, 64)),
            _full_spec((64, 64)), _full_spec((1, 64)), _full_spec((64, 64)),
        ],
        out_specs=[
            pl.BlockSpec((_NB, 64), lambda i: (i, 0)),
            pl.BlockSpec((_NB, 128), lambda i: (i, 0)),
        ],
        out_shape=[
            jax.ShapeDtypeStruct((_N, 64), jnp.float32),
            jax.ShapeDtypeStruct((_N, 128), jnp.float32),
        ],
    )(a1t, x, ga, gx, g0.reshape(1, 64), g1w, g1b.reshape(1, 64), g2w,
      g2b.reshape(1, 64), wd2, b02.reshape(1, 64), ws2)


def _k4_body(at_ref, h_ref, x_ref, goal_ref, ga_ref, gh_ref, g0_ref,
             g1w_ref, g1b_ref, g2w_ref, g2b_ref, qf_ref, qg_ref, q0_ref,
             q1w_ref, q1b_ref, q2w_ref, q2b_ref, out_ref):
    a = at_ref[...].reshape(64, _NB)
    a = jnp.where(a == -jnp.inf, 0.0, a)
    f = _relu(_dot_ct(a, ga_ref[...]) + _dot(h_ref[...], gh_ref[...])
              + g0_ref[...])
    f = _relu(_dot(f, g1w_ref[...]) + g1b_ref[...])
    feat = _dot(f, g2w_ref[...]) + g2b_ref[...]
    g = _relu(_dot(feat, qf_ref[...]) + _dot(goal_ref[...], qg_ref[...])
              + q0_ref[...])
    g = _relu(_dot(g, q1w_ref[...]) + q1b_ref[...])
    g = _dot(g, q2w_ref[...]) + q2b_ref[...]
    gains = 2.0 / (1.0 + jnp.exp(-g)) + 0.2
    x = x_ref[...]
    goal = goal_ref[...]
    s0 = x[:, 0:1] - goal[:, 0:1]
    s1 = x[:, 1:2] - goal[:, 1:2]
    ax = -(gains[:, 0:1] * s0 + gains[:, 1:2] * x[:, 2:3])
    ay = -(gains[:, 2:3] * s1 + gains[:, 3:4] * x[:, 3:4])
    out_ref[...] = jnp.concatenate([ax, ay], axis=1)


def _head(a2t, h, x, goal, ga, gh, g0, g1w, g1b, g2w, g2b,
          qf, qg, q0, q1w, q1b, q2w, q2b):
    nblk = (_N + _NB - 1) // _NB
    return pl.pallas_call(
        _k4_body,
        grid=(nblk,),
        in_specs=[
            pl.BlockSpec((64, _NB // 128, 128), lambda i: (0, i, 0)),
            pl.BlockSpec((_NB, 64), lambda i: (i, 0)),
            pl.BlockSpec((_NB, 4), lambda i: (i, 0)),
            pl.BlockSpec((_NB, 8), lambda i: (i, 0)),
            _full_spec((64, 64)), _full_spec((64, 64)), _full_spec((1, 64)),
            _full_spec((64, 64)), _full_spec((1, 64)), _full_spec((64, 64)),
            _full_spec((1, 64)),
            _full_spec((64, 64)), _full_spec((8, 64)), _full_spec((1, 64)),
            _full_spec((64, 64)), _full_spec((1, 64)), _full_spec((64, 4)),
            _full_spec((1, 4)),
        ],
        out_specs=pl.BlockSpec((_NB, 2), lambda i: (i, 0)),
        out_shape=jax.ShapeDtypeStruct((_N, 2), jnp.float32),
    )(a2t, h, x, goal, ga, gh, g0.reshape(1, 64), g1w, g1b.reshape(1, 64),
      g2w, g2b.reshape(1, 64), qf, qg, q0.reshape(1, 64), q1w,
      q1b.reshape(1, 64), q2w, q2b.reshape(1, 4))


def _sc_gather_sum(table, dst, src, win):
    e = dst.shape[0]
    info = plsc.get_sparse_core_info()
    nc, ns = info.num_cores, info.num_subcores
    per_w = e // (nc * ns)
    nwin = per_w // win
    mesh = plsc.VectorSubcoreMesh(core_axis_name="c", subcore_axis_name="s")

    @functools.partial(
        pl.kernel,
        out_type=jax.ShapeDtypeStruct((e, 64), jnp.float32),
        mesh=mesh,
        compiler_params=pltpu.CompilerParams(use_tc_tiling_on_sc=True, needs_layout_passes=False),
        scratch_types=[
            pltpu.VMEM((win,), jnp.int32),
            pltpu.VMEM((win,), jnp.int32),
            pltpu.VMEM((win, 128), jnp.float32),
            pltpu.VMEM((win, 128), jnp.float32),
            pltpu.VMEM((win, 64), jnp.float32),
            pltpu.SemaphoreType.DMA,
            pltpu.SemaphoreType.DMA,
        ],
    )
    def gather_k(table_hbm, dst_hbm, src_hbm, out_hbm, idxd_v, idxs_v,
                 rowd_v, rows_v, sum_v, semd, sems):
        wid = lax.axis_index("s") * nc + lax.axis_index("c")
        base = wid * per_w

        def body(t, carry):
            off = base + t * win
            pltpu.sync_copy(dst_hbm.at[pl.ds(off, win)], idxd_v)
            pltpu.sync_copy(src_hbm.at[pl.ds(off, win)], idxs_v)
            cpd = pltpu.async_copy(table_hbm.at[idxd_v], rowd_v, semd)
            cps = pltpu.async_copy(table_hbm.at[idxs_v], rows_v, sems)
            cpd.wait()
            cps.wait()

            def row(i, rcarry):
                for c in range(4):
                    sum_v[i, pl.ds(c * 16, 16)] = (
                        rowd_v[i, pl.ds(c * 16, 16)]
                        + rows_v[i, pl.ds(64 + c * 16, 16)])
                return rcarry

            lax.fori_loop(0, win, row, 0)
            pltpu.sync_copy(sum_v, out_hbm.at[pl.ds(off, win)])
            return carry

        lax.fori_loop(0, nwin, body, 0)

    return gather_k(table, dst, src)


def _sc_segmax(msg, dst, win_slabs):
    nslab = msg.shape[1]
    nwin = nslab // win_slabs
    tail = nslab - nwin * win_slabs
    info = plsc.get_sparse_core_info()
    nc, ns = info.num_cores, info.num_subcores
    mesh = plsc.VectorSubcoreMesh(core_axis_name="c", subcore_axis_name="s")

    @functools.partial(
        pl.kernel,
        out_type=jax.ShapeDtypeStruct((64, _NS, 128), jnp.float32),
        mesh=mesh,
        compiler_params=pltpu.CompilerParams(use_tc_tiling_on_sc=True, needs_layout_passes=False),
        scratch_types=[
            pltpu.VMEM((_NS, 128), jnp.float32),
            pltpu.VMEM((_NS, 128), jnp.float32),
            pltpu.VMEM((win_slabs * 128,), jnp.int32),
            pltpu.VMEM((win_slabs, 128), jnp.float32),
            pltpu.VMEM((win_slabs, 128), jnp.float32),
        ],
    )
    def seg_k(msg_hbm, dst_hbm, out_hbm, acc0, acc1, idx_v, v0, v1):
        wid = lax.axis_index("s") * nc + lax.axis_index("c")
        r0 = 2 * wid
        neg = jnp.full((16,), -jnp.inf, jnp.float32)

        def init(i, carry):
            acc0[i >> 3, pl.ds((i & 7) * 16, 16)] = neg
            acc1[i >> 3, pl.ds((i & 7) * 16, 16)] = neg
            return carry

        lax.fori_loop(0, _NS * 8, init, 0)

        def do_groups(ngrp):
            def group(j, gcarry):
                idx = idx_v[pl.ds(j * 16, 16)]
                a = v0[j >> 3, pl.ds((j & 7) * 16, 16)]
                b = v1[j >> 3, pl.ds((j & 7) * 16, 16)]
                row = lax.shift_right_logical(idx, 7)
                col = lax.bitwise_and(idx, 127)

                c0 = plsc.load_gather(acc0, [row, col])
                c1 = plsc.load_gather(acc1, [row, col])
                n0 = jnp.maximum(a, c0)
                n1 = jnp.maximum(b, c1)
                plsc.store_scatter(acc0, [row, col], n0, mask=a > c0)
                plsc.store_scatter(acc1, [row, col], n1, mask=b > c1)
                r0_ = plsc.load_gather(acc0, [row, col])
                r1_ = plsc.load_gather(acc1, [row, col])
                lost = jnp.logical_or(r0_ < n0, r1_ < n1)

                @pl.when(jnp.any(lost))
                def _fixup():
                    lanes = lax.broadcasted_iota(jnp.int32, (16,), 0)

                    def lane(i, lcarry):
                        li = jnp.full((16,), i, jnp.int32)
                        ri = _permute(row, li)
                        ci = _permute(col, li)
                        ai = _permute(a, li)
                        bi = _permute(b, li)
                        one = lanes == i
                        f0 = plsc.load_gather(acc0, [ri, ci])
                        f1 = plsc.load_gather(acc1, [ri, ci])
                        plsc.store_scatter(acc0, [ri, ci],
                                           jnp.maximum(ai, f0), mask=one)
                        plsc.store_scatter(acc1, [ri, ci],
                                           jnp.maximum(bi, f1), mask=one)
                        return lcarry

                    lax.fori_loop(0, 16, lane, 0)

                return gcarry

            lax.fori_loop(0, ngrp, group, 0)

        def window(t, carry):
            soff = t * win_slabs
            pltpu.sync_copy(dst_hbm.at[pl.ds(soff * 128, win_slabs * 128)],
                            idx_v)
            pltpu.sync_copy(msg_hbm.at[r0, pl.ds(soff, win_slabs)], v0)
            pltpu.sync_copy(msg_hbm.at[r0 + 1, pl.ds(soff, win_slabs)], v1)
            do_groups(win_slabs * 8)
            return carry

        lax.fori_loop(0, nwin, window, 0)

        if tail:
            soff = nwin * win_slabs
            pltpu.sync_copy(dst_hbm.at[pl.ds(soff * 128, tail * 128)],
                            idx_v.at[pl.ds(0, tail * 128)])
            pltpu.sync_copy(msg_hbm.at[r0, pl.ds(soff, tail)],
                            v0.at[pl.ds(0, tail)])
            pltpu.sync_copy(msg_hbm.at[r0 + 1, pl.ds(soff, tail)],
                            v1.at[pl.ds(0, tail)])
            do_groups(tail * 8)

        pltpu.sync_copy(acc0, out_hbm.at[r0])
        pltpu.sync_copy(acc1, out_hbm.at[r0 + 1])

    return seg_k(msg, dst)


def kernel(x, edge_attr, edge_index, goal, params):
    src = edge_index[0]
    dst = edge_index[1]

    (w0, b0), (w1, b1), (w2, b2) = params['phi1']
    (g10, g1b0), (g11, g1b1), (g12, g1b2) = params['gamma1']
    (p0, pb0), (p1, pb1), (p2, pb2) = params['phi2']
    (g20, g2b0), (g21, g2b1), (g22, g2b2) = params['gamma2']
    (q0, qb0), (q1, qb1), (q2, qb2) = params['gains']

    t1 = _prep_table1(x, w0[0:4], w0[4:8], b0)
    g1 = _sc_gather_sum(t1, dst, src, win=200)
    msg1_t = _edge_mlp(g1, edge_attr, w0[8:12], w1, b1, w2, b2)
    a1_t = _sc_segmax(msg1_t, dst, win_slabs=32)
    h, t2 = _gamma1_proj(a1_t, x, g10[:64], g10[64:68], g1b0, g11, g1b1,
                         g12, g1b2, p0[:64], pb0, p0[64:128])

    g2 = _sc_gather_sum(t2, dst, src, win=200)
    msg2_t = _edge_mlp(g2, edge_attr, p0[128:132], p1, pb1, p2, pb2)
    a2_t = _sc_segmax(msg2_t, dst, win_slabs=32)
    return _head(a2_t, h, x, goal, g20[:64], g20[64:128], g2b0, g21, g2b1,
                 g22, g2b2, q0[:64], q0[64:72], qb0, q1, qb1, q2, qb2)


# speedup vs baseline: 1.8939x; 1.3078x over previous
import functools

import jax
import jax.numpy as jnp
from jax import lax
from jax.experimental import pallas as pl
from jax.experimental.pallas import tpu as pltpu
from jax.experimental.pallas import tpu_sc as plsc

_N = 50000
_E = 800000
_EB = 4096
_NB = 2048
_NS = 391


def _relu(v):
    return jnp.maximum(v, 0.0)


def _dot(a, b):
    return lax.dot_general(a, b, (((1,), (0,)), ((), ())),
                           preferred_element_type=jnp.float32)


def _dot_t(w, h):
    return lax.dot_general(w, h, (((0,), (1,)), ((), ())),
                           preferred_element_type=jnp.float32)


def _dot_ct(a_t, w):
    return lax.dot_general(a_t, w, (((0,), (0,)), ((), ())),
                           preferred_element_type=jnp.float32)


def _full_spec(shape):
    return pl.BlockSpec(shape, lambda i, r=len(shape): (0,) * r)


def _permute(v, idx):
    return lax.gather(
        v, idx[:, None],
        lax.GatherDimensionNumbers(offset_dims=(), collapsed_slice_dims=(0,),
                                   start_index_map=(0,)),
        (1,), mode=lax.GatherScatterMode.PROMISE_IN_BOUNDS)


def _prep_body(x_ref, wd_ref, ws_ref, b0_ref, t_ref):
    x = x_ref[...]
    t_ref[...] = jnp.concatenate(
        [_dot(x, wd_ref[...]) + b0_ref[...], _dot(x, ws_ref[...])], axis=1)


def _prep_table1(x, wd, ws, b0):
    nblk = (_N + _NB - 1) // _NB
    return pl.pallas_call(
        _prep_body,
        grid=(nblk,),
        in_specs=[
            pl.BlockSpec((_NB, 4), lambda i: (i, 0)),
            _full_spec((4, 64)), _full_spec((4, 64)), _full_spec((1, 64)),
        ],
        out_specs=pl.BlockSpec((_NB, 128), lambda i: (i, 0)),
        out_shape=jax.ShapeDtypeStruct((_N, 128), jnp.float32),
    )(x, wd, ws, b0.reshape(1, 64))


def _edge_body(g_ref, ea_ref, we_ref, w1_ref, b1_ref, w2_ref, b2t_ref,
               out_ref):
    h = _relu(g_ref[...] + _dot(ea_ref[...], we_ref[...]))
    h = _relu(_dot(h, w1_ref[...]) + b1_ref[...])
    m = _dot_t(w2_ref[...], h) + b2t_ref[...]
    out_ref[...] = m.reshape(64, _EB // 128, 128)


def _edge_mlp(g, ea, we, w1, b1, w2, b2):
    nblk = (_E + _EB - 1) // _EB
    return pl.pallas_call(
        _edge_body,
        grid=(nblk,),
        in_specs=[
            pl.BlockSpec((_EB, 64), lambda i: (i, 0)),
            pl.BlockSpec((_EB, 4), lambda i: (i, 0)),
            _full_spec((4, 64)),
            _full_spec((64, 64)), _full_spec((1, 64)),
            _full_spec((64, 64)), _full_spec((64, 1)),
        ],
        out_specs=pl.BlockSpec((64, _EB // 128, 128), lambda i: (0, i, 0)),
        out_shape=jax.ShapeDtypeStruct((64, _E // 128, 128), jnp.float32),
    )(g, ea, we, w1, b1.reshape(1, 64), w2, b2.reshape(64, 1))


def _k2_body(at_ref, x_ref, ga_ref, gx_ref, g0_ref, g1w_ref, g1b_ref,
             g2w_ref, g2b_ref, wd2_ref, b02_ref, ws2_ref, h_ref, t_ref):
    a = at_ref[...].reshape(64, _NB)
    a = jnp.where(a == -jnp.inf, 0.0, a)
    h = _relu(_dot_ct(a, ga_ref[...]) + _dot(x_ref[...], gx_ref[...])
              + g0_ref[...])
    h = _relu(_dot(h, g1w_ref[...]) + g1b_ref[...])
    h = _relu(_dot(h, g2w_ref[...]) + g2b_ref[...])
    h_ref[...] = h
    t_ref[...] = jnp.concatenate(
        [_dot(h, wd2_ref[...]) + b02_ref[...], _dot(h, ws2_ref[...])],
        axis=1)


def _gamma1_proj(a1t, x, ga, gx, g0, g1w, g1b, g2w, g2b, wd2, b02, ws2):
    nblk = (_N + _NB - 1) // _NB
    return pl.pallas_call(
        _k2_body,
        grid=(nblk,),
        in_specs=[
            pl.BlockSpec((64, _NB // 128, 128), lambda i: (0, i, 0)),
            pl.BlockSpec((_NB, 4), lambda i: (i, 0)),
            _full_spec((64, 64)), _full_spec((4, 64)), _full_spec((1, 64)),
            _full_spec((64, 64)), _full_spec((1, 64)),
            _full_spec((64, 64)), _full_spec((1, 64)),
            _full_spec((64, 64)), _full_spec((1, 64)), _full_spec((64, 64)),
        ],
        out_specs=[
            pl.BlockSpec((_NB, 64), lambda i: (i, 0)),
            pl.BlockSpec((_NB, 128), lambda i: (i, 0)),
        ],
        out_shape=[
            jax.ShapeDtypeStruct((_N, 64), jnp.float32),
            jax.ShapeDtypeStruct((_N, 128), jnp.float32),
        ],
    )(a1t, x, ga, gx, g0.reshape(1, 64), g1w, g1b.reshape(1, 64), g2w,
      g2b.reshape(1, 64), wd2, b02.reshape(1, 64), ws2)


def _k4_body(at_ref, h_ref, x_ref, goal_ref, ga_ref, gh_ref, g0_ref,
             g1w_ref, g1b_ref, g2w_ref, g2b_ref, qf_ref, qg_ref, q0_ref,
             q1w_ref, q1b_ref, q2w_ref, q2b_ref, out_ref):
    a = at_ref[...].reshape(64, _NB)
    a = jnp.where(a == -jnp.inf, 0.0, a)
    f = _relu(_dot_ct(a, ga_ref[...]) + _dot(h_ref[...], gh_ref[...])
              + g0_ref[...])
    f = _relu(_dot(f, g1w_ref[...]) + g1b_ref[...])
    feat = _dot(f, g2w_ref[...]) + g2b_ref[...]
    g = _relu(_dot(feat, qf_ref[...]) + _dot(goal_ref[...], qg_ref[...])
              + q0_ref[...])
    g = _relu(_dot(g, q1w_ref[...]) + q1b_ref[...])
    g = _dot(g, q2w_ref[...]) + q2b_ref[...]
    gains = 2.0 / (1.0 + jnp.exp(-g)) + 0.2
    x = x_ref[...]
    goal = goal_ref[...]
    s0 = x[:, 0:1] - goal[:, 0:1]
    s1 = x[:, 1:2] - goal[:, 1:2]
    ax = -(gains[:, 0:1] * s0 + gains[:, 1:2] * x[:, 2:3])
    ay = -(gains[:, 2:3] * s1 + gains[:, 3:4] * x[:, 3:4])
    out_ref[...] = jnp.concatenate([ax, ay], axis=1)


def _head(a2t, h, x, goal, ga, gh, g0, g1w, g1b, g2w, g2b,
          qf, qg, q0, q1w, q1b, q2w, q2b):
    nblk = (_N + _NB - 1) // _NB
    return pl.pallas_call(
        _k4_body,
        grid=(nblk,),
        in_specs=[
            pl.BlockSpec((64, _NB // 128, 128), lambda i: (0, i, 0)),
            pl.BlockSpec((_NB, 64), lambda i: (i, 0)),
            pl.BlockSpec((_NB, 4), lambda i: (i, 0)),
            pl.BlockSpec((_NB, 8), lambda i: (i, 0)),
            _full_spec((64, 64)), _full_spec((64, 64)), _full_spec((1, 64)),
            _full_spec((64, 64)), _full_spec((1, 64)), _full_spec((64, 64)),
            _full_spec((1, 64)),
            _full_spec((64, 64)), _full_spec((8, 64)), _full_spec((1, 64)),
            _full_spec((64, 64)), _full_spec((1, 64)), _full_spec((64, 4)),
            _full_spec((1, 4)),
        ],
        out_specs=pl.BlockSpec((_NB, 2), lambda i: (i, 0)),
        out_shape=jax.ShapeDtypeStruct((_N, 2), jnp.float32),
    )(a2t, h, x, goal, ga, gh, g0.reshape(1, 64), g1w, g1b.reshape(1, 64),
      g2w, g2b.reshape(1, 64), qf, qg, q0.reshape(1, 64), q1w,
      q1b.reshape(1, 64), q2w, q2b.reshape(1, 4))


def _sc_gather_sum(table, dst, src, win):
    e = dst.shape[0]
    info = plsc.get_sparse_core_info()
    nc, ns = info.num_cores, info.num_subcores
    per_w = e // (nc * ns)
    nwin = per_w // win
    mesh = plsc.VectorSubcoreMesh(core_axis_name="c", subcore_axis_name="s")

    @functools.partial(
        pl.kernel,
        out_type=jax.ShapeDtypeStruct((e, 64), jnp.float32),
        mesh=mesh,
        compiler_params=pltpu.CompilerParams(use_tc_tiling_on_sc=True, needs_layout_passes=False),
        scratch_types=[
            pltpu.VMEM((win,), jnp.int32),
            pltpu.VMEM((win,), jnp.int32),
            pltpu.VMEM((win, 128), jnp.float32),
            pltpu.VMEM((win, 128), jnp.float32),
            pltpu.VMEM((win, 64), jnp.float32),
            pltpu.SemaphoreType.DMA,
            pltpu.SemaphoreType.DMA,
        ],
    )
    def gather_k(table_hbm, dst_hbm, src_hbm, out_hbm, idxd_v, idxs_v,
                 rowd_v, rows_v, sum_v, semd, sems):
        wid = lax.axis_index("s") * nc + lax.axis_index("c")
        base = wid * per_w

        def body(t, carry):
            off = base + t * win
            pltpu.sync_copy(dst_hbm.at[pl.ds(off, win)], idxd_v)
            pltpu.sync_copy(src_hbm.at[pl.ds(off, win)], idxs_v)
            cpd = pltpu.async_copy(table_hbm.at[idxd_v], rowd_v, semd)
            cps = pltpu.async_copy(table_hbm.at[idxs_v], rows_v, sems)
            cpd.wait()
            cps.wait()

            def row(i, rcarry):
                for c in range(4):
                    sum_v[i, pl.ds(c * 16, 16)] = (
                        rowd_v[i, pl.ds(c * 16, 16)]
                        + rows_v[i, pl.ds(64 + c * 16, 16)])
                return rcarry

            lax.fori_loop(0, win, row, 0)
            pltpu.sync_copy(sum_v, out_hbm.at[pl.ds(off, win)])
            return carry

        lax.fori_loop(0, nwin, body, 0)

    return gather_k(table, dst, src)


def _sc_segmax(msg, dst, win_slabs):
    nslab = msg.shape[1]
    nwin = nslab // win_slabs
    tail = nslab - nwin * win_slabs
    info = plsc.get_sparse_core_info()
    nc, ns = info.num_cores, info.num_subcores
    mesh = plsc.VectorSubcoreMesh(core_axis_name="c", subcore_axis_name="s")

    @functools.partial(
        pl.kernel,
        out_type=jax.ShapeDtypeStruct((64, _NS, 128), jnp.float32),
        mesh=mesh,
        compiler_params=pltpu.CompilerParams(use_tc_tiling_on_sc=True, needs_layout_passes=False),
        scratch_types=[
            pltpu.VMEM((_NS, 128), jnp.float32),
            pltpu.VMEM((_NS, 128), jnp.float32),
            pltpu.VMEM((win_slabs * 128,), jnp.int32),
            pltpu.VMEM((win_slabs, 128), jnp.float32),
            pltpu.VMEM((win_slabs, 128), jnp.float32),
        ],
    )
    def seg_k(msg_hbm, dst_hbm, out_hbm, acc0, acc1, idx_v, v0, v1):
        wid = lax.axis_index("s") * nc + lax.axis_index("c")
        r0 = 2 * wid
        neg = jnp.full((16,), -jnp.inf, jnp.float32)

        def init(i, carry):
            acc0[i >> 3, pl.ds((i & 7) * 16, 16)] = neg
            acc1[i >> 3, pl.ds((i & 7) * 16, 16)] = neg
            return carry

        lax.fori_loop(0, _NS * 8, init, 0)

        def do_groups(ngrp):
            def group(j, gcarry):
                raw = idx_v[pl.ds(j * 16, 16)]
                a = v0[j >> 3, pl.ds((j & 7) * 16, 16)]
                b = v1[j >> 3, pl.ds((j & 7) * 16, 16)]
                idx = lax.bitwise_and(raw, 0xFFFF)
                row = lax.shift_right_logical(idx, 7)
                col = lax.bitwise_and(idx, 127)

                c0 = plsc.load_gather(acc0, [row, col])
                c1 = plsc.load_gather(acc1, [row, col])
                plsc.store_scatter(acc0, [row, col], jnp.maximum(a, c0))
                plsc.store_scatter(acc1, [row, col], jnp.maximum(b, c1))

                @pl.when(jnp.any(raw > 0xFFFF))
                def _fixup():
                    lanes = lax.broadcasted_iota(jnp.int32, (16,), 0)

                    def lane(i, lcarry):
                        li = jnp.full((16,), i, jnp.int32)
                        ri = _permute(row, li)
                        ci = _permute(col, li)
                        ai = _permute(a, li)
                        bi = _permute(b, li)
                        one = lanes == i
                        f0 = plsc.load_gather(acc0, [ri, ci])
                        f1 = plsc.load_gather(acc1, [ri, ci])
                        plsc.store_scatter(acc0, [ri, ci],
                                           jnp.maximum(ai, f0), mask=one)
                        plsc.store_scatter(acc1, [ri, ci],
                                           jnp.maximum(bi, f1), mask=one)
                        return lcarry

                    lax.fori_loop(0, 16, lane, 0)

                return gcarry

            lax.fori_loop(0, ngrp, group, 0, unroll=4)

        def window(t, carry):
            soff = t * win_slabs
            pltpu.sync_copy(dst_hbm.at[pl.ds(soff * 128, win_slabs * 128)],
                            idx_v)
            pltpu.sync_copy(msg_hbm.at[r0, pl.ds(soff, win_slabs)], v0)
            pltpu.sync_copy(msg_hbm.at[r0 + 1, pl.ds(soff, win_slabs)], v1)
            do_groups(win_slabs * 8)
            return carry

        lax.fori_loop(0, nwin, window, 0)

        if tail:
            soff = nwin * win_slabs
            pltpu.sync_copy(dst_hbm.at[pl.ds(soff * 128, tail * 128)],
                            idx_v.at[pl.ds(0, tail * 128)])
            pltpu.sync_copy(msg_hbm.at[r0, pl.ds(soff, tail)],
                            v0.at[pl.ds(0, tail)])
            pltpu.sync_copy(msg_hbm.at[r0 + 1, pl.ds(soff, tail)],
                            v1.at[pl.ds(0, tail)])
            do_groups(tail * 8)

        pltpu.sync_copy(acc0, out_hbm.at[r0])
        pltpu.sync_copy(acc1, out_hbm.at[r0 + 1])

    return seg_k(msg, dst)


def kernel(x, edge_attr, edge_index, goal, params):
    src = edge_index[0]
    dst = edge_index[1]

    (w0, b0), (w1, b1), (w2, b2) = params['phi1']
    (g10, g1b0), (g11, g1b1), (g12, g1b2) = params['gamma1']
    (p0, pb0), (p1, pb1), (p2, pb2) = params['phi2']
    (g20, g2b0), (g21, g2b1), (g22, g2b2) = params['gamma2']
    (q0, qb0), (q1, qb1), (q2, qb2) = params['gains']

    d16 = dst.reshape(_E // 16, 16)
    dup = (d16[:, :, None] == d16[:, None, :]).sum(axis=(1, 2)) > 16
    dstf = (d16 | (dup.astype(jnp.int32) << 16)[:, None]).reshape(_E)

    t1 = _prep_table1(x, w0[0:4], w0[4:8], b0)
    g1 = _sc_gather_sum(t1, dst, src, win=200)
    msg1_t = _edge_mlp(g1, edge_attr, w0[8:12], w1, b1, w2, b2)
    a1_t = _sc_segmax(msg1_t, dstf, win_slabs=32)
    h, t2 = _gamma1_proj(a1_t, x, g10[:64], g10[64:68], g1b0, g11, g1b1,
                         g12, g1b2, p0[:64], pb0, p0[64:128])

    g2 = _sc_gather_sum(t2, dst, src, win=200)
    msg2_t = _edge_mlp(g2, edge_attr, p0[128:132], p1, pb1, p2, pb2)
    a2_t = _sc_segmax(msg2_t, dstf, win_slabs=32)
    return _head(a2_t, h, x, goal, g20[:64], g20[64:128], g2b0, g21, g2b1,
                 g22, g2b2, q0[:64], q0[64:72], qb0, q1, qb1, q2, qb2)


# speedup vs baseline: 3.5305x; 1.8642x over previous
import functools

import jax
import jax.numpy as jnp
from jax import lax
from jax.experimental import pallas as pl
from jax.experimental.pallas import tpu as pltpu
from jax.experimental.pallas import tpu_sc as plsc

_N = 50000
_E = 800000
_EB = 4096
_NB = 2048
_NS = 391


def _relu(v):
    return jnp.maximum(v, 0.0)


def _dot(a, b):
    return lax.dot_general(a, b, (((1,), (0,)), ((), ())),
                           preferred_element_type=jnp.float32)


def _dot_t(w, h):
    return lax.dot_general(w, h, (((0,), (1,)), ((), ())),
                           preferred_element_type=jnp.float32)


def _dot_ct(a_t, w):
    return lax.dot_general(a_t, w, (((0,), (0,)), ((), ())),
                           preferred_element_type=jnp.float32)


def _full_spec(shape):
    return pl.BlockSpec(shape, lambda i, r=len(shape): (0,) * r)


def _permute(v, idx):
    return lax.gather(
        v, idx[:, None],
        lax.GatherDimensionNumbers(offset_dims=(), collapsed_slice_dims=(0,),
                                   start_index_map=(0,)),
        (1,), mode=lax.GatherScatterMode.PROMISE_IN_BOUNDS)


def _prep_body(x_ref, wd_ref, ws_ref, b0_ref, t_ref):
    x = x_ref[...]
    t_ref[...] = jnp.concatenate(
        [_dot(x, wd_ref[...]) + b0_ref[...], _dot(x, ws_ref[...])], axis=1)


def _prep_table1(x, wd, ws, b0):
    nblk = (_N + _NB - 1) // _NB
    return pl.pallas_call(
        _prep_body,
        grid=(nblk,),
        in_specs=[
            pl.BlockSpec((_NB, 4), lambda i: (i, 0)),
            _full_spec((4, 64)), _full_spec((4, 64)), _full_spec((1, 64)),
        ],
        out_specs=pl.BlockSpec((_NB, 128), lambda i: (i, 0)),
        out_shape=jax.ShapeDtypeStruct((_N, 128), jnp.float32),
    )(x, wd, ws, b0.reshape(1, 64))


def _edge_body(g_ref, ea_ref, we_ref, w1_ref, b1_ref, w2_ref, b2t_ref,
               out_ref):
    h = _relu(g_ref[...] + _dot(ea_ref[...], we_ref[...]))
    h = _relu(_dot(h, w1_ref[...]) + b1_ref[...])
    m = _dot_t(w2_ref[...], h) + b2t_ref[...]
    out_ref[...] = m.reshape(64, _EB // 128, 128)


def _edge_mlp(g, ea, we, w1, b1, w2, b2):
    nblk = (_E + _EB - 1) // _EB
    return pl.pallas_call(
        _edge_body,
        grid=(nblk,),
        in_specs=[
            pl.BlockSpec((_EB, 64), lambda i: (i, 0)),
            pl.BlockSpec((_EB, 4), lambda i: (i, 0)),
            _full_spec((4, 64)),
            _full_spec((64, 64)), _full_spec((1, 64)),
            _full_spec((64, 64)), _full_spec((64, 1)),
        ],
        out_specs=pl.BlockSpec((64, _EB // 128, 128), lambda i: (0, i, 0)),
        out_shape=jax.ShapeDtypeStruct((64, _E // 128, 128), jnp.float32),
    )(g, ea, we, w1, b1.reshape(1, 64), w2, b2.reshape(64, 1))


def _k2_body(at_ref, x_ref, ga_ref, gx_ref, g0_ref, g1w_ref, g1b_ref,
             g2w_ref, g2b_ref, wd2_ref, b02_ref, ws2_ref, h_ref, t_ref):
    a = at_ref[...].reshape(64, _NB)
    a = jnp.where(a == -jnp.inf, 0.0, a)
    h = _relu(_dot_ct(a, ga_ref[...]) + _dot(x_ref[...], gx_ref[...])
              + g0_ref[...])
    h = _relu(_dot(h, g1w_ref[...]) + g1b_ref[...])
    h = _relu(_dot(h, g2w_ref[...]) + g2b_ref[...])
    h_ref[...] = h
    t_ref[...] = jnp.concatenate(
        [_dot(h, wd2_ref[...]) + b02_ref[...], _dot(h, ws2_ref[...])],
        axis=1)


def _gamma1_proj(a1t, x, ga, gx, g0, g1w, g1b, g2w, g2b, wd2, b02, ws2):
    nblk = (_N + _NB - 1) // _NB
    return pl.pallas_call(
        _k2_body,
        grid=(nblk,),
        in_specs=[
            pl.BlockSpec((64, _NB // 128, 128), lambda i: (0, i, 0)),
            pl.BlockSpec((_NB, 4), lambda i: (i, 0)),
            _full_spec((64, 64)), _full_spec((4, 64)), _full_spec((1, 64)),
            _full_spec((64, 64)), _full_spec((1, 64)),
            _full_spec((64, 64)), _full_spec((1, 64)),
            _full_spec((64, 64)), _full_spec((1, 64)), _full_spec((64, 64)),
        ],
        out_specs=[
            pl.BlockSpec((_NB, 64), lambda i: (i, 0)),
            pl.BlockSpec((_NB, 128), lambda i: (i, 0)),
        ],
        out_shape=[
            jax.ShapeDtypeStruct((_N, 64), jnp.float32),
            jax.ShapeDtypeStruct((_N, 128), jnp.float32),
        ],
    )(a1t, x, ga, gx, g0.reshape(1, 64), g1w, g1b.reshape(1, 64), g2w,
      g2b.reshape(1, 64), wd2, b02.reshape(1, 64), ws2)


def _k4_body(at_ref, h_ref, x_ref, goal_ref, ga_ref, gh_ref, g0_ref,
             g1w_ref, g1b_ref, g2w_ref, g2b_ref, qf_ref, qg_ref, q0_ref,
             q1w_ref, q1b_ref, q2w_ref, q2b_ref, out_ref):
    a = at_ref[...].reshape(64, _NB)
    a = jnp.where(a == -jnp.inf, 0.0, a)
    f = _relu(_dot_ct(a, ga_ref[...]) + _dot(h_ref[...], gh_ref[...])
              + g0_ref[...])
    f = _relu(_dot(f, g1w_ref[...]) + g1b_ref[...])
    feat = _dot(f, g2w_ref[...]) + g2b_ref[...]
    g = _relu(_dot(feat, qf_ref[...]) + _dot(goal_ref[...], qg_ref[...])
              + q0_ref[...])
    g = _relu(_dot(g, q1w_ref[...]) + q1b_ref[...])
    g = _dot(g, q2w_ref[...]) + q2b_ref[...]
    gains = 2.0 / (1.0 + jnp.exp(-g)) + 0.2
    x = x_ref[...]
    goal = goal_ref[...]
    s0 = x[:, 0:1] - goal[:, 0:1]
    s1 = x[:, 1:2] - goal[:, 1:2]
    ax = -(gains[:, 0:1] * s0 + gains[:, 1:2] * x[:, 2:3])
    ay = -(gains[:, 2:3] * s1 + gains[:, 3:4] * x[:, 3:4])
    out_ref[...] = jnp.concatenate([ax, ay], axis=1)


def _head(a2t, h, x, goal, ga, gh, g0, g1w, g1b, g2w, g2b,
          qf, qg, q0, q1w, q1b, q2w, q2b):
    nblk = (_N + _NB - 1) // _NB
    return pl.pallas_call(
        _k4_body,
        grid=(nblk,),
        in_specs=[
            pl.BlockSpec((64, _NB // 128, 128), lambda i: (0, i, 0)),
            pl.BlockSpec((_NB, 64), lambda i: (i, 0)),
            pl.BlockSpec((_NB, 4), lambda i: (i, 0)),
            pl.BlockSpec((_NB, 8), lambda i: (i, 0)),
            _full_spec((64, 64)), _full_spec((64, 64)), _full_spec((1, 64)),
            _full_spec((64, 64)), _full_spec((1, 64)), _full_spec((64, 64)),
            _full_spec((1, 64)),
            _full_spec((64, 64)), _full_spec((8, 64)), _full_spec((1, 64)),
            _full_spec((64, 64)), _full_spec((1, 64)), _full_spec((64, 4)),
            _full_spec((1, 4)),
        ],
        out_specs=pl.BlockSpec((_NB, 2), lambda i: (i, 0)),
        out_shape=jax.ShapeDtypeStruct((_N, 2), jnp.float32),
    )(a2t, h, x, goal, ga, gh, g0.reshape(1, 64), g1w, g1b.reshape(1, 64),
      g2w, g2b.reshape(1, 64), qf, qg, q0.reshape(1, 64), q1w,
      q1b.reshape(1, 64), q2w, q2b.reshape(1, 4))


def _sc_gather_sum(table, dst, src, win):
    e = dst.shape[0]
    info = plsc.get_sparse_core_info()
    nc, ns = info.num_cores, info.num_subcores
    per_w = e // (nc * ns)
    nwin = per_w // win
    mesh = plsc.VectorSubcoreMesh(core_axis_name="c", subcore_axis_name="s")

    assert nwin >= 3

    @functools.partial(
        pl.kernel,
        out_type=jax.ShapeDtypeStruct((e, 64), jnp.float32),
        mesh=mesh,
        compiler_params=pltpu.CompilerParams(use_tc_tiling_on_sc=True, needs_layout_passes=False),
        scratch_types=[
            pltpu.VMEM((win,), jnp.int32),
            pltpu.VMEM((win,), jnp.int32),
            pltpu.VMEM((win,), jnp.int32),
            pltpu.VMEM((win,), jnp.int32),
            pltpu.VMEM((win, 128), jnp.float32),
            pltpu.VMEM((win, 128), jnp.float32),
            pltpu.VMEM((win, 128), jnp.float32),
            pltpu.VMEM((win, 128), jnp.float32),
            pltpu.VMEM((win, 64), jnp.float32),
            pltpu.SemaphoreType.DMA,
            pltpu.SemaphoreType.DMA,
            pltpu.SemaphoreType.DMA,
            pltpu.SemaphoreType.DMA,
            pltpu.SemaphoreType.DMA,
            pltpu.SemaphoreType.DMA,
        ],
    )
    def gather_k(table_hbm, dst_hbm, src_hbm, out_hbm, idxd0, idxd1,
                 idxs0, idxs1, rowd0, rowd1, rows0, rows1, sum0,
                 si0, si1, sg0, sg1, so0, so1):
        wid = lax.axis_index("s") * nc + lax.axis_index("c")
        base = wid * per_w
        idxd, idxs = [idxd0, idxd1], [idxs0, idxs1]
        rowd, rows = [rowd0, rowd1], [rows0, rows1]
        sums = [sum0, sum0]
        sis, sgs, sos = [si0, si1], [sg0, sg1], [so0, so1]

        def issue_idx(t, p):
            off = base + t * win
            pltpu.async_copy(dst_hbm.at[pl.ds(off, win)], idxd[p], sis[p])
            pltpu.async_copy(src_hbm.at[pl.ds(off, win)], idxs[p], sis[p])

        def wait_idx(p):
            pltpu.make_async_copy(dst_hbm.at[pl.ds(0, win)], idxd[p],
                                  sis[p]).wait()
            pltpu.make_async_copy(src_hbm.at[pl.ds(0, win)], idxs[p],
                                  sis[p]).wait()

        def issue_gather(p):
            pltpu.async_copy(table_hbm.at[idxd[p]], rowd[p], sgs[p])
            pltpu.async_copy(table_hbm.at[idxs[p]], rows[p], sgs[p])

        def wait_gather(p):
            pltpu.make_async_copy(table_hbm.at[idxd[p]], rowd[p],
                                  sgs[p]).wait()
            pltpu.make_async_copy(table_hbm.at[idxs[p]], rows[p],
                                  sgs[p]).wait()

        def issue_out(t, p):
            off = base + t * win
            pltpu.async_copy(sums[p], out_hbm.at[pl.ds(off, win)], sos[p])

        def wait_out(p):
            pltpu.make_async_copy(sums[p], out_hbm.at[pl.ds(0, win)],
                                  sos[p]).wait()

        def add_loop(p):
            def row(i, rcarry):
                for c in range(4):
                    sums[p][i, pl.ds(c * 16, 16)] = (
                        rowd[p][i, pl.ds(c * 16, 16)]
                        + rows[p][i, pl.ds(64 + c * 16, 16)])
                return rcarry

            lax.fori_loop(0, win, row, 0, unroll=2)

        def do_iter(t, p):
            t = jnp.int32(t)
            q = 1 - p
            wait_gather(p)

            @pl.when(t + 2 < nwin)
            def _pf_idx():
                issue_idx(t + 2, p)

            @pl.when(t + 1 < nwin)
            def _pf_gather():
                wait_idx(q)
                issue_gather(q)

            @pl.when(t >= 1)
            def _drain_out():
                wait_out(q)

            add_loop(p)
            issue_out(t, p)

        issue_idx(0, 0)
        wait_idx(0)
        issue_gather(0)
        issue_idx(1, 1)

        def pair(tp, carry):
            do_iter(2 * tp, 0)
            do_iter(2 * tp + 1, 1)
            return carry

        lax.fori_loop(0, nwin // 2, pair, 0)
        if nwin % 2:
            do_iter(nwin - 1, 0)
        wait_out((nwin - 1) % 2)

    return gather_k(table, dst, src)


def _sc_segmax(msg, dst, win_slabs):
    nslab = msg.shape[1]
    nwin = nslab // win_slabs
    tail = nslab - nwin * win_slabs
    info = plsc.get_sparse_core_info()
    nc, ns = info.num_cores, info.num_subcores
    mesh = plsc.VectorSubcoreMesh(core_axis_name="c", subcore_axis_name="s")

    @functools.partial(
        pl.kernel,
        out_type=jax.ShapeDtypeStruct((64, _NS, 128), jnp.float32),
        mesh=mesh,
        compiler_params=pltpu.CompilerParams(use_tc_tiling_on_sc=True, needs_layout_passes=False),
        scratch_types=[
            pltpu.VMEM((_NS, 128), jnp.float32),
            pltpu.VMEM((_NS, 128), jnp.float32),
            pltpu.VMEM((win_slabs * 128,), jnp.int32),
            pltpu.VMEM((win_slabs * 128,), jnp.int32),
            pltpu.VMEM((win_slabs, 128), jnp.float32),
            pltpu.VMEM((win_slabs, 128), jnp.float32),
            pltpu.VMEM((win_slabs, 128), jnp.float32),
            pltpu.VMEM((win_slabs, 128), jnp.float32),
            pltpu.SemaphoreType.DMA,
            pltpu.SemaphoreType.DMA,
        ],
    )
    def seg_k(msg_hbm, dst_hbm, out_hbm, acc0, acc1, ixa, ixb, v0a, v0b,
              v1a, v1b, sma, smb):
        wid = lax.axis_index("s") * nc + lax.axis_index("c")
        r0 = 2 * wid
        neg = jnp.full((16,), -jnp.inf, jnp.float32)
        ixs, v0s, v1s, sems = [ixa, ixb], [v0a, v0b], [v1a, v1b], [sma, smb]

        def issue(t, p, nsl):
            soff = t * win_slabs
            pltpu.async_copy(dst_hbm.at[pl.ds(soff * 128, nsl * 128)],
                             ixs[p].at[pl.ds(0, nsl * 128)], sems[p])
            pltpu.async_copy(msg_hbm.at[r0, pl.ds(soff, nsl)],
                             v0s[p].at[pl.ds(0, nsl)], sems[p])
            pltpu.async_copy(msg_hbm.at[r0 + 1, pl.ds(soff, nsl)],
                             v1s[p].at[pl.ds(0, nsl)], sems[p])

        def wait_set(p, nsl):
            pltpu.make_async_copy(dst_hbm.at[pl.ds(0, nsl * 128)],
                                  ixs[p].at[pl.ds(0, nsl * 128)],
                                  sems[p]).wait()
            pltpu.make_async_copy(msg_hbm.at[r0, pl.ds(0, nsl)],
                                  v0s[p].at[pl.ds(0, nsl)], sems[p]).wait()
            pltpu.make_async_copy(msg_hbm.at[r0, pl.ds(0, nsl)],
                                  v1s[p].at[pl.ds(0, nsl)], sems[p]).wait()

        def init(i, carry):
            acc0[i >> 3, pl.ds((i & 7) * 16, 16)] = neg
            acc1[i >> 3, pl.ds((i & 7) * 16, 16)] = neg
            return carry

        lax.fori_loop(0, _NS * 8, init, 0)

        def do_groups(nslab, p):
            idx_v, v0, v1 = ixs[p], v0s[p], v1s[p]

            def slab(s, gcarry):
                raws, rows, cols, avs, bvs = [], [], [], [], []
                flag = None
                for g in range(8):
                    raw = idx_v[pl.ds((s * 8 + g) * 16, 16)]
                    f = raw > 0xFFFF
                    flag = f if flag is None else jnp.logical_or(flag, f)
                    idx = lax.bitwise_and(raw, 0xFFFF)
                    raws.append(raw)
                    rows.append(lax.shift_right_logical(idx, 7))
                    cols.append(lax.bitwise_and(idx, 127))
                    avs.append(v0[s, pl.ds(g * 16, 16)])
                    bvs.append(v1[s, pl.ds(g * 16, 16)])

                for g in range(8):
                    c0 = plsc.load_gather(acc0, [rows[g], cols[g]])
                    c1 = plsc.load_gather(acc1, [rows[g], cols[g]])
                    plsc.store_scatter(acc0, [rows[g], cols[g]],
                                       jnp.maximum(avs[g], c0))
                    plsc.store_scatter(acc1, [rows[g], cols[g]],
                                       jnp.maximum(bvs[g], c1))

                @pl.when(jnp.any(flag))
                def _fixup():
                    lanes = lax.broadcasted_iota(jnp.int32, (16,), 0)
                    for g in range(8):
                        row, col, a, b = rows[g], cols[g], avs[g], bvs[g]

                        @pl.when(jnp.any(raws[g] > 0xFFFF))
                        def _fix_group(row=row, col=col, a=a, b=b):
                            def lane(i, lcarry):
                                li = jnp.full((16,), i, jnp.int32)
                                ri = _permute(row, li)
                                ci = _permute(col, li)
                                ai = _permute(a, li)
                                bi = _permute(b, li)
                                one = lanes == i
                                f0 = plsc.load_gather(acc0, [ri, ci])
                                f1 = plsc.load_gather(acc1, [ri, ci])
                                plsc.store_scatter(acc0, [ri, ci],
                                                   jnp.maximum(ai, f0),
                                                   mask=one)
                                plsc.store_scatter(acc1, [ri, ci],
                                                   jnp.maximum(bi, f1),
                                                   mask=one)
                                return lcarry

                            lax.fori_loop(0, 16, lane, 0)

                return gcarry

            lax.fori_loop(0, nslab, slab, 0)

        def witer(t, p):
            t = jnp.int32(t)
            q = 1 - p
            wait_set(p, win_slabs)

            @pl.when(t + 1 < nwin)
            def _pf():
                issue(t + 1, q, win_slabs)

            do_groups(win_slabs, p)

        issue(jnp.int32(0), 0, win_slabs)

        def pair(k, carry):
            witer(2 * k, 0)
            witer(2 * k + 1, 1)
            return carry

        lax.fori_loop(0, nwin // 2, pair, 0)
        if nwin % 2:
            witer(nwin - 1, (nwin - 1) % 2)
        if tail:
            soff = nwin * win_slabs
            pltpu.sync_copy(dst_hbm.at[pl.ds(soff * 128, tail * 128)],
                            ixs[0].at[pl.ds(0, tail * 128)])
            pltpu.sync_copy(msg_hbm.at[r0, pl.ds(soff, tail)],
                            v0s[0].at[pl.ds(0, tail)])
            pltpu.sync_copy(msg_hbm.at[r0 + 1, pl.ds(soff, tail)],
                            v1s[0].at[pl.ds(0, tail)])
            do_groups(tail, 0)

        pltpu.sync_copy(acc0, out_hbm.at[r0])
        pltpu.sync_copy(acc1, out_hbm.at[r0 + 1])

    return seg_k(msg, dst)


def kernel(x, edge_attr, edge_index, goal, params):
    src = edge_index[0]
    dst = edge_index[1]

    (w0, b0), (w1, b1), (w2, b2) = params['phi1']
    (g10, g1b0), (g11, g1b1), (g12, g1b2) = params['gamma1']
    (p0, pb0), (p1, pb1), (p2, pb2) = params['phi2']
    (g20, g2b0), (g21, g2b1), (g22, g2b2) = params['gamma2']
    (q0, qb0), (q1, qb1), (q2, qb2) = params['gains']

    d16 = dst.reshape(_E // 16, 16)
    dup = (d16[:, :, None] == d16[:, None, :]).sum(axis=(1, 2)) > 16
    dstf = (d16 | (dup.astype(jnp.int32) << 16)[:, None]).reshape(_E)

    t1 = _prep_table1(x, w0[0:4], w0[4:8], b0)
    g1 = _sc_gather_sum(t1, dst, src, win=200)
    msg1_t = _edge_mlp(g1, edge_attr, w0[8:12], w1, b1, w2, b2)
    a1_t = _sc_segmax(msg1_t, dstf, win_slabs=16)
    h, t2 = _gamma1_proj(a1_t, x, g10[:64], g10[64:68], g1b0, g11, g1b1,
                         g12, g1b2, p0[:64], pb0, p0[64:128])

    g2 = _sc_gather_sum(t2, dst, src, win=200)
    msg2_t = _edge_mlp(g2, edge_attr, p0[128:132], p1, pb1, p2, pb2)
    a2_t = _sc_segmax(msg2_t, dstf, win_slabs=16)
    return _head(a2_t, h, x, goal, g20[:64], g20[64:128], g2b0, g21, g2b1,
                 g22, g2b2, q0[:64], q0[64:72], qb0, q1, qb1, q2, qb2)
